# baseline (speedup 1.0000x reference)
import functools

import jax
import jax.numpy as jnp
from jax import lax
from jax.experimental import pallas as pl
from jax.experimental.pallas import tpu as pltpu
from jax.experimental.pallas import tpu_sc as plsc

_BATCH = 16384
_FIELDS = 26
_DIM = 32
_NW = 32
_NB = _BATCH // _NW
_NG = 3
_TP = _NB + 3

_mesh = plsc.VectorSubcoreMesh(core_axis_name="c", subcore_axis_name="s")


@functools.partial(
    pl.kernel,
    mesh=_mesh,
    out_type=jax.ShapeDtypeStruct((_FIELDS, _DIM, _BATCH), jnp.float32),
    scratch_types=[
        pltpu.VMEM((_FIELDS, _NB), jnp.int32),
        pltpu.VMEM((_NG, _NB, _DIM), jnp.float32),
        pltpu.VMEM((2, _DIM, _TP), jnp.float32),
        [pltpu.SemaphoreType.DMA] * _NG,
        [pltpu.SemaphoreType.DMA] * _NG,
        [pltpu.SemaphoreType.DMA] * 2,
    ],
    compiler_params=pltpu.CompilerParams(
        use_tc_tiling_on_sc=False, needs_layout_passes=False
    ),
)
def _gather(tok_hbm, table_hbm, out_hbm, idx_v, g_v, t_v, gsems, gsems2, tsems):
    wid = lax.axis_index("s") * 2 + lax.axis_index("c")
    b0 = wid * _NB

    pltpu.sync_copy(tok_hbm.at[:, pl.ds(b0, _NB)], idx_v)

    _H = _NB // 2

    def start_gather(f, gb):
        pltpu.async_copy(
            table_hbm.at[idx_v.at[f, pl.ds(0, _H)]],
            g_v.at[gb, pl.ds(0, _H)],
            gsems[gb],
        )
        pltpu.async_copy(
            table_hbm.at[idx_v.at[f, pl.ds(_H, _H)]],
            g_v.at[gb, pl.ds(_H, _H)],
            gsems2[gb],
        )

    for gb in range(_NG):
        start_gather(gb, gb)

    lanes = lax.iota(jnp.int32, 16)

    def transpose_block(gb, tb):
        def body(b, carry):
            col = jnp.full((16,), b, jnp.int32)
            lo = g_v[gb, b, pl.ds(0, 16)]
            hi = g_v[gb, b, pl.ds(16, 16)]
            plsc.store_scatter(t_v.at[tb], [lanes, col], lo)
            plsc.store_scatter(t_v.at[tb], [lanes + 16, col], hi)
            return carry

        lax.fori_loop(0, _NB, body, 0)

    for f in range(_FIELDS):
        gb = f % _NG
        tb = f % 2
        pltpu.make_async_copy(
            table_hbm.at[idx_v.at[f, pl.ds(0, _H)]],
            g_v.at[gb, pl.ds(0, _H)],
            gsems[gb],
        ).wait()
        pltpu.make_async_copy(
            table_hbm.at[idx_v.at[f, pl.ds(_H, _H)]],
            g_v.at[gb, pl.ds(_H, _H)],
            gsems2[gb],
        ).wait()
        if f >= 2:
            pltpu.make_async_copy(
                t_v.at[tb, :, pl.ds(0, _NB)],
                out_hbm.at[f - 2, :, pl.ds(b0, _NB)],
                tsems[tb],
            ).wait()
        transpose_block(gb, tb)
        nf = f + _NG
        if nf < _FIELDS:
            start_gather(nf, gb)
        pltpu.async_copy(
            t_v.at[tb, :, pl.ds(0, _NB)],
            out_hbm.at[f, :, pl.ds(b0, _NB)],
            tsems[tb],
        )

    for f in (_FIELDS - 2, _FIELDS - 1):
        tb = f % 2
        pltpu.make_async_copy(
            t_v.at[tb, :, pl.ds(0, _NB)],
            out_hbm.at[f, :, pl.ds(b0, _NB)],
            tsems[tb],
        ).wait()


def kernel(token_ids, embeddings):
    tok_t = token_ids.T.astype(jnp.int32)
    out = _gather(tok_t, embeddings)
    return jnp.transpose(out, (2, 0, 1))


# speedup vs baseline: 1.0047x; 1.0047x over previous
import functools

import jax
import jax.numpy as jnp
from jax import lax
from jax.experimental import pallas as pl
from jax.experimental.pallas import tpu as pltpu
from jax.experimental.pallas import tpu_sc as plsc

_BATCH = 16384
_FIELDS = 26
_DIM = 32
_NW = 32
_NB = _BATCH // _NW
_NG = 3
_TP = _NB + 3

_mesh = plsc.VectorSubcoreMesh(core_axis_name="c", subcore_axis_name="s")


@functools.partial(
    pl.kernel,
    mesh=_mesh,
    out_type=jax.ShapeDtypeStruct((_FIELDS, _DIM, _BATCH), jnp.float32),
    scratch_types=[
        pltpu.VMEM((_FIELDS, _NB), jnp.int32),
        pltpu.VMEM((_NG, _NB, _DIM), jnp.float32),
        pltpu.VMEM((2, _DIM, _TP), jnp.float32),
        [pltpu.SemaphoreType.DMA] * _NG,
        [pltpu.SemaphoreType.DMA] * 2,
    ],
    compiler_params=pltpu.CompilerParams(
        use_tc_tiling_on_sc=False, needs_layout_passes=False
    ),
)
def _gather(tok_hbm, table_hbm, out_hbm, idx_v, g_v, t_v, gsems, tsems):
    wid = lax.axis_index("s") * 2 + lax.axis_index("c")
    b0 = wid * _NB

    pltpu.sync_copy(tok_hbm.at[:, pl.ds(b0, _NB)], idx_v)

    def start_gather(f, gb):
        pltpu.async_copy(table_hbm.at[idx_v.at[f]], g_v.at[gb], gsems[gb])

    for gb in range(_NG):
        start_gather(gb, gb)

    lanes = lax.iota(jnp.int32, 16)

    def transpose_block(gb, tb):
        def body(i, carry):
            b = i * 4
            for u in range(4):
                col = jnp.full((16,), b + u, jnp.int32)
                lo = g_v[gb, b + u, pl.ds(0, 16)]
                hi = g_v[gb, b + u, pl.ds(16, 16)]
                plsc.store_scatter(t_v.at[tb], [lanes, col], lo)
                plsc.store_scatter(t_v.at[tb], [lanes + 16, col], hi)
            return carry

        lax.fori_loop(0, _NB // 4, body, 0)

    for f in range(_FIELDS):
        gb = f % _NG
        tb = f % 2
        pltpu.make_async_copy(
            table_hbm.at[idx_v.at[f]], g_v.at[gb], gsems[gb]
        ).wait()
        if f >= 2:
            pltpu.make_async_copy(
                t_v.at[tb, :, pl.ds(0, _NB)],
                out_hbm.at[f - 2, :, pl.ds(b0, _NB)],
                tsems[tb],
            ).wait()
        transpose_block(gb, tb)
        nf = f + _NG
        if nf < _FIELDS:
            start_gather(nf, gb)
        pltpu.async_copy(
            t_v.at[tb, :, pl.ds(0, _NB)],
            out_hbm.at[f, :, pl.ds(b0, _NB)],
            tsems[tb],
        )

    for f in (_FIELDS - 2, _FIELDS - 1):
        tb = f % 2
        pltpu.make_async_copy(
            t_v.at[tb, :, pl.ds(0, _NB)],
            out_hbm.at[f, :, pl.ds(b0, _NB)],
            tsems[tb],
        ).wait()


def kernel(token_ids, embeddings):
    tok_t = token_ids.T.astype(jnp.int32)
    out = _gather(tok_t, embeddings)
    return jnp.transpose(out, (2, 0, 1))


# speedup vs baseline: 1.0063x; 1.0016x over previous
import functools

import jax
import jax.numpy as jnp
from jax import lax
from jax.experimental import pallas as pl
from jax.experimental.pallas import tpu as pltpu
from jax.experimental.pallas import tpu_sc as plsc

_BATCH = 16384
_FIELDS = 26
_DIM = 32
_NW = 32
_NB = _BATCH // _NW
_NG = 4
_TP = _NB + 3

_mesh = plsc.VectorSubcoreMesh(core_axis_name="c", subcore_axis_name="s")


@functools.partial(
    pl.kernel,
    mesh=_mesh,
    out_type=jax.ShapeDtypeStruct((_FIELDS, _DIM, _BATCH), jnp.float32),
    scratch_types=[
        pltpu.VMEM((_FIELDS, _NB), jnp.int32),
        pltpu.VMEM((_NG, _NB, _DIM), jnp.float32),
        pltpu.VMEM((2, _DIM, _TP), jnp.float32),
        [pltpu.SemaphoreType.DMA] * _NG,
        [pltpu.SemaphoreType.DMA] * 2,
    ],
    compiler_params=pltpu.CompilerParams(
        use_tc_tiling_on_sc=False, needs_layout_passes=False
    ),
)
def _gather(tok_hbm, table_hbm, out_hbm, idx_v, g_v, t_v, gsems, tsems):
    wid = lax.axis_index("s") * 2 + lax.axis_index("c")
    b0 = wid * _NB

    pltpu.sync_copy(tok_hbm.at[:, pl.ds(b0, _NB)], idx_v)

    def start_gather(f, gb):
        pltpu.async_copy(table_hbm.at[idx_v.at[f]], g_v.at[gb], gsems[gb])

    for gb in range(_NG):
        start_gather(gb, gb)

    lanes = lax.iota(jnp.int32, 16)

    def transpose_block(gb, tb):
        def body(i, carry):
            b = i * 4
            for u in range(4):
                col = jnp.full((16,), b + u, jnp.int32)
                lo = g_v[gb, b + u, pl.ds(0, 16)]
                hi = g_v[gb, b + u, pl.ds(16, 16)]
                plsc.store_scatter(t_v.at[tb], [lanes, col], lo)
                plsc.store_scatter(t_v.at[tb], [lanes + 16, col], hi)
            return carry

        lax.fori_loop(0, _NB // 4, body, 0)

    for f in range(_FIELDS):
        gb = f % _NG
        tb = f % 2
        pltpu.make_async_copy(
            table_hbm.at[idx_v.at[f]], g_v.at[gb], gsems[gb]
        ).wait()
        if f >= 2:
            pltpu.make_async_copy(
                t_v.at[tb, :, pl.ds(0, _NB)],
                out_hbm.at[f - 2, :, pl.ds(b0, _NB)],
                tsems[tb],
            ).wait()
        transpose_block(gb, tb)
        nf = f + _NG
        if nf < _FIELDS:
            start_gather(nf, gb)
        pltpu.async_copy(
            t_v.at[tb, :, pl.ds(0, _NB)],
            out_hbm.at[f, :, pl.ds(b0, _NB)],
            tsems[tb],
        )

    for f in (_FIELDS - 2, _FIELDS - 1):
        tb = f % 2
        pltpu.make_async_copy(
            t_v.at[tb, :, pl.ds(0, _NB)],
            out_hbm.at[f, :, pl.ds(b0, _NB)],
            tsems[tb],
        ).wait()


def kernel(token_ids, embeddings):
    tok_t = token_ids.T.astype(jnp.int32)
    out = _gather(tok_t, embeddings)
    return jnp.transpose(out, (2, 0, 1))
